# baseline (speedup 1.0000x reference)
import functools

import jax
import jax.numpy as jnp
from jax import lax
from jax.experimental import pallas as pl
from jax.experimental.pallas import tpu as pltpu
from jax.experimental.pallas import tpu_sc as plsc


def _make_router_body(T, E, BLK, NB, CH):
    def body(x_ref, rw_ref, pos0_ref, pos1_ref, g0_ref, g1_ref,
             blke_ref):
        x = x_ref[...]
        rw = rw_ref[...]
        logits = jnp.dot(x, rw, preferred_element_type=jnp.float32)

        e_iota = lax.broadcasted_iota(jnp.int32, (T, E), 1)
        m1 = jnp.max(logits, axis=1, keepdims=True)
        i1 = jnp.argmax(logits, axis=1).astype(jnp.int32)
        oh1 = e_iota == i1[:, None]
        logits2 = jnp.where(oh1, -jnp.inf, logits)
        m2 = jnp.max(logits2, axis=1, keepdims=True)
        i2 = jnp.argmax(logits2, axis=1).astype(jnp.int32)
        oh2 = e_iota == i2[:, None]
        g1v = 1.0 / (1.0 + jnp.exp(m2 - m1))
        g2v = 1.0 - g1v

        oh1f = oh1.astype(jnp.float32)
        oh2f = oh2.astype(jnp.float32)
        ohf = oh1f + oh2f

        r_i = lax.broadcasted_iota(jnp.int32, (CH, CH), 0)
        c_i = lax.broadcasted_iota(jnp.int32, (CH, CH), 1)
        ltri = (r_i > c_i).astype(jnp.float32)
        carry = jnp.zeros((1, E), jnp.float32)
        parts = []
        for k in range(T // CH):
            blk = ohf[k * CH:(k + 1) * CH]
            parts.append(
                jnp.dot(ltri, blk, preferred_element_type=jnp.float32) + carry)
            carry = carry + jnp.sum(blk, axis=0, keepdims=True)
        ex = jnp.concatenate(parts, axis=0)
        counts = carry

        pc = jnp.floor((counts + (BLK - 1)) / BLK) * BLK
        m8r = lax.broadcasted_iota(jnp.int32, (E, E), 0)
        m8c = lax.broadcasted_iota(jnp.int32, (E, E), 1)
        mstrict = (m8r < m8c).astype(jnp.float32)
        po = jnp.dot(pc, mstrict, preferred_element_type=jnp.float32)

        base0 = jnp.sum(po * oh1f, axis=1)
        base1 = jnp.sum(po * oh2f, axis=1)
        rank0 = jnp.sum(ex * oh1f, axis=1)
        rank1 = jnp.sum(ex * oh2f, axis=1)
        pos0_ref[0, :] = (base0 + rank0).astype(jnp.int32)
        pos1_ref[0, :] = (base1 + rank1).astype(jnp.int32)
        g0_ref[...] = jnp.broadcast_to(g1v, (T, 16))
        g1_ref[...] = jnp.broadcast_to(g2v, (T, 16))

        ends = po + pc
        bstart = (lax.broadcasted_iota(jnp.int32, (NB, E), 0) * BLK
                  ).astype(jnp.float32)
        full = (ends <= bstart).astype(jnp.int32)
        blkvec = jnp.minimum(jnp.sum(full, axis=1), E - 1)
        used = (ends[0, E - 1] / BLK).astype(jnp.int32)
        blke_ref[0, :] = jnp.concatenate(
            [blkvec, jnp.broadcast_to(used, (8,))])

    return body


def _router_call(x, router_w, T, E, BLK, NB, CH, interpret=False):
    return pl.pallas_call(
        _make_router_body(T, E, BLK, NB, CH),
        out_shape=[
            jax.ShapeDtypeStruct((1, T), jnp.int32),
            jax.ShapeDtypeStruct((1, T), jnp.int32),
            jax.ShapeDtypeStruct((T, 16), jnp.float32),
            jax.ShapeDtypeStruct((T, 16), jnp.float32),
            jax.ShapeDtypeStruct((1, NB + 8), jnp.int32),
        ],
        interpret=interpret,
    )(x, router_w)


def _make_mlp1_body():
    def body(blke_ref, xs_ref, w1_ref, b1_ref, h_ref):
        i = pl.program_id(0)
        nblk = blke_ref[blke_ref.shape[0] - 8]

        @pl.when(i < nblk)
        def _():
            xb = xs_ref[...].astype(jnp.bfloat16)
            h = jnp.dot(xb, w1_ref[0],
                        preferred_element_type=jnp.float32)
            h_ref[...] = jax.nn.gelu(h + b1_ref[0]).astype(jnp.bfloat16)

    return body


def _make_mlp2_body():
    def body(blke_ref, h_ref, w2_ref, b2_ref, ys_ref):
        i = pl.program_id(0)
        nblk = blke_ref[blke_ref.shape[0] - 8]

        @pl.when(i < nblk)
        def _():
            ys_ref[...] = (
                jnp.dot(h_ref[...], w2_ref[0],
                        preferred_element_type=jnp.float32)
                + b2_ref[0])

    return body


def _mlp_call(blke, xs, w1, b1, w2, b2, S, D, F, BLK, NB, interpret=False):
    grid_spec1 = pltpu.PrefetchScalarGridSpec(
        num_scalar_prefetch=1,
        grid=(NB,),
        in_specs=[
            pl.BlockSpec((BLK, D), lambda i, be: (i, 0)),
            pl.BlockSpec((1, D, F), lambda i, be: (be[i], 0, 0)),
            pl.BlockSpec((1, 1, F), lambda i, be: (be[i], 0, 0)),
        ],
        out_specs=pl.BlockSpec((BLK, F), lambda i, be: (i, 0)),
    )
    h = pl.pallas_call(
        _make_mlp1_body(),
        grid_spec=grid_spec1,
        out_shape=jax.ShapeDtypeStruct((S, F), jnp.bfloat16),
        interpret=interpret,
    )(blke, xs, w1, b1.reshape(b1.shape[0], 1, b1.shape[1]))
    grid_spec2 = pltpu.PrefetchScalarGridSpec(
        num_scalar_prefetch=1,
        grid=(NB,),
        in_specs=[
            pl.BlockSpec((BLK, F), lambda i, be: (i, 0)),
            pl.BlockSpec((1, F, D), lambda i, be: (be[i], 0, 0)),
            pl.BlockSpec((1, 1, D), lambda i, be: (be[i], 0, 0)),
        ],
        out_specs=pl.BlockSpec((BLK, D), lambda i, be: (i, 0)),
    )
    return pl.pallas_call(
        _make_mlp2_body(),
        grid_spec=grid_spec2,
        out_shape=jax.ShapeDtypeStruct((S, D), jnp.float32),
        interpret=interpret,
    )(blke, h, w2, b2.reshape(b2.shape[0], 1, b2.shape[1]))


_NC = 2
_NS = 16
_NW = _NC * _NS


def _make_dispatch(T, D, S, CB):
    mesh = plsc.VectorSubcoreMesh(core_axis_name="c", subcore_axis_name="s")
    TW = T // _NW
    NK = TW // CB

    @functools.partial(
        pl.kernel, mesh=mesh,
        out_type=jax.ShapeDtypeStruct((S, D), jnp.float32),
        scratch_types=[
            pltpu.VMEM((CB, D), jnp.float32),
            pltpu.VMEM((CB, D), jnp.float32),
            pltpu.VMEM((CB,), jnp.int32),
            pltpu.VMEM((CB,), jnp.int32),
            pltpu.VMEM((CB,), jnp.int32),
            pltpu.VMEM((CB,), jnp.int32),
            pltpu.SemaphoreType.DMA,
            pltpu.SemaphoreType.DMA,
        ],
    )
    def dispatch(x_hbm, p0_hbm, p1_hbm, xs_hbm, xbuf0, xbuf1,
                 i0a, i1a, i0b, i1b, sem0, sem1):
        wid = lax.axis_index("s") * _NC + lax.axis_index("c")
        base = wid * TW
        xbufs = (xbuf0, xbuf1)
        ibufs = ((i0a, i1a), (i0b, i1b))
        sems = (sem0, sem1)
        pend = [None] * NK
        for k in range(NK):
            p = k % 2
            if k >= 2:
                for cp in pend[k - 2]:
                    cp.wait()
            cb = base + k * CB
            pltpu.sync_copy(x_hbm.at[pl.ds(cb, CB)], xbufs[p])
            pltpu.sync_copy(p0_hbm.at[pl.ds(cb, CB)], ibufs[p][0])
            pltpu.sync_copy(p1_hbm.at[pl.ds(cb, CB)], ibufs[p][1])
            pend[k] = [
                pltpu.async_copy(xbufs[p], xs_hbm.at[ibufs[p][0]], sems[p]),
                pltpu.async_copy(xbufs[p], xs_hbm.at[ibufs[p][1]], sems[p]),
            ]
        for k in range(max(NK - 2, 0), NK):
            for cp in pend[k]:
                cp.wait()

    return dispatch


def _make_combine(T, D, S, CD):
    mesh = plsc.VectorSubcoreMesh(core_axis_name="c", subcore_axis_name="s")
    TW = T // _NW
    NJ = D // 16

    NK = TW // CD

    @functools.partial(
        pl.kernel, mesh=mesh,
        out_type=jax.ShapeDtypeStruct((T, D), jnp.float32),
        scratch_types=[
            pltpu.VMEM((CD,), jnp.int32),
            pltpu.VMEM((CD,), jnp.int32),
            pltpu.VMEM((CD,), jnp.int32),
            pltpu.VMEM((CD,), jnp.int32),
            pltpu.VMEM((CD, D), jnp.float32),
            pltpu.VMEM((CD, D), jnp.float32),
            pltpu.VMEM((CD, D), jnp.float32),
            pltpu.VMEM((CD, D), jnp.float32),
            pltpu.VMEM((CD * 16,), jnp.float32),
            pltpu.VMEM((CD * 16,), jnp.float32),
            pltpu.VMEM((CD * 16,), jnp.float32),
            pltpu.VMEM((CD * 16,), jnp.float32),
            pltpu.SemaphoreType.DMA,
            pltpu.SemaphoreType.DMA,
        ],
    )
    def combine(ys_hbm, p0_hbm, p1_hbm, g0_hbm, g1_hbm, out_hbm,
                i0a, i1a, i0b, i1b, r0a, r1a, r0b, r1b,
                g0a, g1a, g0b, g1b, sem0, sem1):
        wid = lax.axis_index("s") * _NC + lax.axis_index("c")
        base = wid * TW
        ibufs = ((i0a, i1a), (i0b, i1b))
        rbufs = ((r0a, r1a), (r0b, r1b))
        gbufs = ((g0a, g1a), (g0b, g1b))
        sems = (sem0, sem1)
        pend = [None] * NK

        def stage(k):
            p = k % 2
            cb = base + k * CD
            pltpu.sync_copy(p0_hbm.at[pl.ds(cb, CD)], ibufs[p][0])
            pltpu.sync_copy(p1_hbm.at[pl.ds(cb, CD)], ibufs[p][1])
            pltpu.sync_copy(g0_hbm.at[pl.ds(cb * 16, CD * 16)], gbufs[p][0])
            pltpu.sync_copy(g1_hbm.at[pl.ds(cb * 16, CD * 16)], gbufs[p][1])
            pend[k] = [
                pltpu.async_copy(ys_hbm.at[ibufs[p][0]], rbufs[p][0], sems[p]),
                pltpu.async_copy(ys_hbm.at[ibufs[p][1]], rbufs[p][1], sems[p]),
            ]

        def flush(k):
            p = k % 2
            cb = base + k * CD
            for cp in pend[k]:
                cp.wait()
            r0buf, r1buf = rbufs[p]
            g0buf, g1buf = gbufs[p]

            def row_fn(i, _):
                gv0 = g0buf[pl.ds(i * 16, 16)]
                gv1 = g1buf[pl.ds(i * 16, 16)]

                def col_fn(j, _):
                    for u in range(4):
                        s = pl.ds((j * 4 + u) * 16, 16)
                        r0buf[i, s] = gv0 * r0buf[i, s] + gv1 * r1buf[i, s]
                    return 0

                lax.fori_loop(0, NJ // 4, col_fn, 0)
                return 0

            lax.fori_loop(0, CD, row_fn, 0)
            pltpu.sync_copy(r0buf, out_hbm.at[pl.ds(cb, CD)])

        stage(0)
        for k in range(1, NK):
            stage(k)
            flush(k - 1)
        flush(NK - 1)

    return combine


def kernel(x, router_w, w1, b1, w2, b2):
    T, D = x.shape
    E = router_w.shape[1]
    F = w1.shape[2]
    BLK = 256
    NB = 2 * T // BLK + E
    S = NB * BLK
    CH = 256

    pos0r, pos1r, g0, g1, blke = _router_call(
        x, router_w, T, E, BLK, NB, CH)
    pos0 = pos0r.reshape(T)
    pos1 = pos1r.reshape(T)

    xs = _make_dispatch(T, D, S, CB=32)(x, pos0, pos1)

    ys = _mlp_call(blke.reshape(NB + 8), xs, w1, b1, w2, b2,
                   S, D, F, BLK, NB)

    out = _make_combine(T, D, S, CD=16)(
        ys, pos0, pos1, g0.reshape(T * 16), g1.reshape(T * 16))
    return out


# speedup vs baseline: 1.0573x; 1.0573x over previous
import functools

import jax
import jax.numpy as jnp
from jax import lax
from jax.experimental import pallas as pl
from jax.experimental.pallas import tpu as pltpu
from jax.experimental.pallas import tpu_sc as plsc


def _make_router_body(T, E, BLK, NB, CH):
    def body(x_ref, rw_ref, pos0_ref, pos1_ref, g0_ref, g1_ref,
             blke_ref):
        x = x_ref[...]
        rw = rw_ref[...]
        logits = jnp.dot(x, rw, preferred_element_type=jnp.float32)

        e_iota = lax.broadcasted_iota(jnp.int32, (T, E), 1)
        m1 = jnp.max(logits, axis=1, keepdims=True)
        i1 = jnp.argmax(logits, axis=1).astype(jnp.int32)
        oh1 = e_iota == i1[:, None]
        logits2 = jnp.where(oh1, -jnp.inf, logits)
        m2 = jnp.max(logits2, axis=1, keepdims=True)
        i2 = jnp.argmax(logits2, axis=1).astype(jnp.int32)
        oh2 = e_iota == i2[:, None]
        g1v = 1.0 / (1.0 + jnp.exp(m2 - m1))
        g2v = 1.0 - g1v

        oh1f = oh1.astype(jnp.float32)
        oh2f = oh2.astype(jnp.float32)
        ohf = oh1f + oh2f

        r_i = lax.broadcasted_iota(jnp.int32, (CH, CH), 0)
        c_i = lax.broadcasted_iota(jnp.int32, (CH, CH), 1)
        ltri = (r_i > c_i).astype(jnp.float32)
        carry = jnp.zeros((1, E), jnp.float32)
        parts = []
        for k in range(T // CH):
            blk = ohf[k * CH:(k + 1) * CH]
            parts.append(
                jnp.dot(ltri, blk, preferred_element_type=jnp.float32) + carry)
            carry = carry + jnp.sum(blk, axis=0, keepdims=True)
        ex = jnp.concatenate(parts, axis=0)
        counts = carry

        pc = jnp.floor((counts + (BLK - 1)) / BLK) * BLK
        m8r = lax.broadcasted_iota(jnp.int32, (E, E), 0)
        m8c = lax.broadcasted_iota(jnp.int32, (E, E), 1)
        mstrict = (m8r < m8c).astype(jnp.float32)
        po = jnp.dot(pc, mstrict, preferred_element_type=jnp.float32)

        base0 = jnp.sum(po * oh1f, axis=1)
        base1 = jnp.sum(po * oh2f, axis=1)
        rank0 = jnp.sum(ex * oh1f, axis=1)
        rank1 = jnp.sum(ex * oh2f, axis=1)
        pos0_ref[0, :] = (base0 + rank0).astype(jnp.int32)
        pos1_ref[0, :] = (base1 + rank1).astype(jnp.int32)
        g0_ref[...] = jnp.broadcast_to(g1v, (T, 16))
        g1_ref[...] = jnp.broadcast_to(g2v, (T, 16))

        ends = po + pc
        bstart = (lax.broadcasted_iota(jnp.int32, (NB, E), 0) * BLK
                  ).astype(jnp.float32)
        full = (ends <= bstart).astype(jnp.int32)
        blkvec = jnp.minimum(jnp.sum(full, axis=1), E - 1)
        used = (ends[0, E - 1] / BLK).astype(jnp.int32)
        blke_ref[0, :] = jnp.concatenate(
            [blkvec, jnp.broadcast_to(used, (8,))])

    return body


def _router_call(x, router_w, T, E, BLK, NB, CH, interpret=False):
    return pl.pallas_call(
        _make_router_body(T, E, BLK, NB, CH),
        out_shape=[
            jax.ShapeDtypeStruct((1, T), jnp.int32),
            jax.ShapeDtypeStruct((1, T), jnp.int32),
            jax.ShapeDtypeStruct((T, 16), jnp.float32),
            jax.ShapeDtypeStruct((T, 16), jnp.float32),
            jax.ShapeDtypeStruct((1, NB + 8), jnp.int32),
        ],
        interpret=interpret,
    )(x, router_w)


def _make_mlp1_body():
    def body(blke_ref, xs_ref, w1_ref, b1_ref, h_ref):
        i = pl.program_id(0)
        nblk = blke_ref[blke_ref.shape[0] - 8]

        @pl.when(i < nblk)
        def _():
            xb = xs_ref[...].astype(jnp.bfloat16)
            h = jnp.dot(xb, w1_ref[0],
                        preferred_element_type=jnp.float32)
            h_ref[...] = jax.nn.gelu(h + b1_ref[0]).astype(jnp.bfloat16)

    return body


def _make_mlp2_body():
    def body(blke_ref, h_ref, w2_ref, b2_ref, ys_ref):
        i = pl.program_id(0)
        nblk = blke_ref[blke_ref.shape[0] - 8]

        @pl.when(i < nblk)
        def _():
            ys_ref[...] = (
                jnp.dot(h_ref[...], w2_ref[0],
                        preferred_element_type=jnp.float32)
                + b2_ref[0])

    return body


def _mlp_call(blke, xs, w1, b1, w2, b2, S, D, F, BLK, NB, interpret=False):
    grid_spec1 = pltpu.PrefetchScalarGridSpec(
        num_scalar_prefetch=1,
        grid=(NB,),
        in_specs=[
            pl.BlockSpec((BLK, D), lambda i, be: (i, 0)),
            pl.BlockSpec((1, D, F), lambda i, be: (be[i], 0, 0)),
            pl.BlockSpec((1, 1, F), lambda i, be: (be[i], 0, 0)),
        ],
        out_specs=pl.BlockSpec((BLK, F), lambda i, be: (i, 0)),
    )
    h = pl.pallas_call(
        _make_mlp1_body(),
        grid_spec=grid_spec1,
        out_shape=jax.ShapeDtypeStruct((S, F), jnp.bfloat16),
        interpret=interpret,
    )(blke, xs, w1, b1.reshape(b1.shape[0], 1, b1.shape[1]))
    grid_spec2 = pltpu.PrefetchScalarGridSpec(
        num_scalar_prefetch=1,
        grid=(NB,),
        in_specs=[
            pl.BlockSpec((BLK, F), lambda i, be: (i, 0)),
            pl.BlockSpec((1, F, D), lambda i, be: (be[i], 0, 0)),
            pl.BlockSpec((1, 1, D), lambda i, be: (be[i], 0, 0)),
        ],
        out_specs=pl.BlockSpec((BLK, D), lambda i, be: (i, 0)),
    )
    return pl.pallas_call(
        _make_mlp2_body(),
        grid_spec=grid_spec2,
        out_shape=jax.ShapeDtypeStruct((S, D), jnp.float32),
        interpret=interpret,
    )(blke, h, w2, b2.reshape(b2.shape[0], 1, b2.shape[1]))


_NC = 2
_NS = 16
_NW = _NC * _NS


def _make_dispatch(T, D, S, CB):
    mesh = plsc.VectorSubcoreMesh(core_axis_name="c", subcore_axis_name="s")
    TW = T // _NW
    NK = TW // CB

    @functools.partial(
        pl.kernel, mesh=mesh,
        out_type=jax.ShapeDtypeStruct((S, D), jnp.float32),
        scratch_types=[
            pltpu.VMEM((CB, D), jnp.float32),
            pltpu.VMEM((CB, D), jnp.float32),
            pltpu.VMEM((CB,), jnp.int32),
            pltpu.VMEM((CB,), jnp.int32),
            pltpu.VMEM((CB,), jnp.int32),
            pltpu.VMEM((CB,), jnp.int32),
            pltpu.SemaphoreType.DMA,
            pltpu.SemaphoreType.DMA,
        ],
    )
    def dispatch(x_hbm, p0_hbm, p1_hbm, xs_hbm, xbuf0, xbuf1,
                 i0a, i1a, i0b, i1b, sem0, sem1):
        wid = lax.axis_index("s") * _NC + lax.axis_index("c")
        base = wid * TW
        xbufs = (xbuf0, xbuf1)
        ibufs = ((i0a, i1a), (i0b, i1b))
        sems = (sem0, sem1)
        pend = [None] * NK
        for k in range(NK):
            p = k % 2
            if k >= 2:
                for cp in pend[k - 2]:
                    cp.wait()
            cb = base + k * CB
            pltpu.sync_copy(x_hbm.at[pl.ds(cb, CB)], xbufs[p])
            pltpu.sync_copy(p0_hbm.at[pl.ds(cb, CB)], ibufs[p][0])
            pltpu.sync_copy(p1_hbm.at[pl.ds(cb, CB)], ibufs[p][1])
            pend[k] = [
                pltpu.async_copy(xbufs[p], xs_hbm.at[ibufs[p][0]], sems[p]),
                pltpu.async_copy(xbufs[p], xs_hbm.at[ibufs[p][1]], sems[p]),
            ]
        for k in range(max(NK - 2, 0), NK):
            for cp in pend[k]:
                cp.wait()

    return dispatch


def _make_combine(T, D, S, CD):
    mesh = plsc.VectorSubcoreMesh(core_axis_name="c", subcore_axis_name="s")
    TW = T // _NW
    NJ = D // 16

    NK = TW // CD

    @functools.partial(
        pl.kernel, mesh=mesh,
        out_type=jax.ShapeDtypeStruct((T, D), jnp.float32),
        scratch_types=[
            pltpu.VMEM((CD,), jnp.int32),
            pltpu.VMEM((CD,), jnp.int32),
            pltpu.VMEM((CD,), jnp.int32),
            pltpu.VMEM((CD,), jnp.int32),
            pltpu.VMEM((CD, D), jnp.float32),
            pltpu.VMEM((CD, D), jnp.float32),
            pltpu.VMEM((CD, D), jnp.float32),
            pltpu.VMEM((CD, D), jnp.float32),
            pltpu.VMEM((CD * 16,), jnp.float32),
            pltpu.VMEM((CD * 16,), jnp.float32),
            pltpu.VMEM((CD * 16,), jnp.float32),
            pltpu.VMEM((CD * 16,), jnp.float32),
            pltpu.SemaphoreType.DMA,
            pltpu.SemaphoreType.DMA,
        ],
    )
    def combine(ys_hbm, p0_hbm, p1_hbm, g0_hbm, g1_hbm, out_hbm,
                i0a, i1a, i0b, i1b, r0a, r1a, r0b, r1b,
                g0a, g1a, g0b, g1b, sem0, sem1):
        wid = lax.axis_index("s") * _NC + lax.axis_index("c")
        base = wid * TW
        ibufs = ((i0a, i1a), (i0b, i1b))
        rbufs = ((r0a, r1a), (r0b, r1b))
        gbufs = ((g0a, g1a), (g0b, g1b))
        sems = (sem0, sem1)
        pend = [None] * NK

        def stage(k):
            p = k % 2
            cb = base + k * CD
            pltpu.sync_copy(p0_hbm.at[pl.ds(cb, CD)], ibufs[p][0])
            pltpu.sync_copy(p1_hbm.at[pl.ds(cb, CD)], ibufs[p][1])
            pltpu.sync_copy(g0_hbm.at[pl.ds(cb * 16, CD * 16)], gbufs[p][0])
            pltpu.sync_copy(g1_hbm.at[pl.ds(cb * 16, CD * 16)], gbufs[p][1])
            pend[k] = [
                pltpu.async_copy(ys_hbm.at[ibufs[p][0]], rbufs[p][0], sems[p]),
                pltpu.async_copy(ys_hbm.at[ibufs[p][1]], rbufs[p][1], sems[p]),
            ]

        def flush(k):
            p = k % 2
            cb = base + k * CD
            for cp in pend[k]:
                cp.wait()
            r0buf, r1buf = rbufs[p]
            g0buf, g1buf = gbufs[p]

            def row_fn(i, _):
                gv0 = g0buf[pl.ds(i * 16, 16)]
                gv1 = g1buf[pl.ds(i * 16, 16)]

                def col_fn(j, _):
                    for u in range(4):
                        s = pl.ds((j * 4 + u) * 16, 16)
                        r0buf[i, s] = gv0 * r0buf[i, s] + gv1 * r1buf[i, s]
                    return 0

                lax.fori_loop(0, NJ // 4, col_fn, 0)
                return 0

            lax.fori_loop(0, CD, row_fn, 0)
            pltpu.sync_copy(r0buf, out_hbm.at[pl.ds(cb, CD)])

        stage(0)
        for k in range(1, NK):
            stage(k)
            flush(k - 1)
        flush(NK - 1)

    return combine


def kernel(x, router_w, w1, b1, w2, b2):
    T, D = x.shape
    E = router_w.shape[1]
    F = w1.shape[2]
    BLK = 512
    NB = 2 * T // BLK + E
    S = NB * BLK
    CH = 256

    pos0r, pos1r, g0, g1, blke = _router_call(
        x, router_w, T, E, BLK, NB, CH)
    pos0 = pos0r.reshape(T)
    pos1 = pos1r.reshape(T)

    xs = _make_dispatch(T, D, S, CB=32)(x, pos0, pos1)

    ys = _mlp_call(blke.reshape(NB + 8), xs, w1, b1, w2, b2,
                   S, D, F, BLK, NB)

    out = _make_combine(T, D, S, CD=16)(
        ys, pos0, pos1, g0.reshape(T * 16), g1.reshape(T * 16))
    return out
